# baseline (speedup 1.0000x reference)
import jax
import jax.numpy as jnp
from jax import lax
from jax.experimental import pallas as pl

_K, _D = 1024, 32
_BLK = 256


def _vq_body(z_ref, s_ref, cb_ref, zq_ref):
    zb = z_ref[...]
    cb = cb_ref[...]
    a = jnp.sum(zb * zb, axis=1, keepdims=True)
    d = jnp.abs(a - s_ref[...])
    m = jnp.min(d, axis=1, keepdims=True)
    iota = lax.broadcasted_iota(jnp.int32, (_BLK, _K), 1)
    idx = jnp.min(jnp.where(d == m, iota, _K), axis=1)
    onehot = (lax.broadcasted_iota(jnp.int32, (_BLK, _K), 1)
              == idx[:, None]).astype(jnp.float32)
    zq_ref[...] = jnp.dot(onehot, cb, preferred_element_type=jnp.float32)


def _vq(zflat, s, codebook):
    n = zflat.shape[0]
    grid = n // _BLK
    return pl.pallas_call(
        _vq_body,
        grid=(grid,),
        in_specs=[
            pl.BlockSpec((_BLK, _D), lambda i: (i, 0)),
            pl.BlockSpec((1, _K), lambda i: (0, 0)),
            pl.BlockSpec((_K, _D), lambda i: (0, 0)),
        ],
        out_specs=pl.BlockSpec((_BLK, _D), lambda i: (i, 0)),
        out_shape=jax.ShapeDtypeStruct((n, _D), jnp.float32),
    )(zflat, s, codebook)


def _conv(x, w, b, stride):
    y = lax.conv_general_dilated(x, w, window_strides=(stride, stride),
                                 padding=((1, 1), (1, 1)),
                                 dimension_numbers=('NCHW', 'OIHW', 'NCHW'))
    return y + b[None, :, None, None]


def _conv_t(x, w, b):
    wf = jnp.flip(w, axis=(2, 3)).transpose(1, 0, 2, 3)
    y = lax.conv_general_dilated(x, wf, window_strides=(1, 1),
                                 padding=((2, 2), (2, 2)),
                                 lhs_dilation=(2, 2),
                                 dimension_numbers=('NCHW', 'OIHW', 'NCHW'))
    return y + b[None, :, None, None]


def kernel(imgs, w1, b1, w2, b2, codebook, wt1, bt1, wt2, bt2):
    h = jax.nn.relu(_conv(imgs, w1, b1, 2))
    z_e = jax.nn.relu(_conv(h, w2, b2, 2))
    n, c, hh, ww = z_e.shape
    zflat = z_e.transpose(0, 2, 3, 1).reshape(-1, _D)
    s = jnp.sum(codebook ** 2, axis=1)
    zq = _vq(zflat, s[None, :], codebook)
    encoded = zq.reshape(n, hh, ww, _D).transpose(0, 3, 1, 2)
    d = jax.nn.relu(_conv_t(encoded, wt1, bt1))
    decoded = jax.nn.relu(_conv_t(d, wt2, bt2))
    return (z_e, encoded, decoded)
